# baseline (speedup 1.0000x reference)
import jax
import jax.numpy as jnp
from jax.experimental import pallas as pl
from jax.experimental.pallas import tpu as pltpu

_ISSUE, _DEV, _FILE = 4000, 2000, 4000
_N = _ISSUE + _DEV + _FILE
_D = 128
_LEAKY = 0.1
_NS = 1
_RS = 200
_R = _NS * _RS
_PREC = jax.lax.Precision.DEFAULT
_SEGS = ((0, _ISSUE), (_ISSUE, _DEV), (_ISSUE + _DEV, _FILE))


def _lk(x):
    return jnp.where(x >= 0, x, _LEAKY * x)


def _layer_body(first, *refs):
    adj_ks, refs = refs[:_NS], refs[_NS:]
    if first:
        ih, dh, fh, lat, tem, hyp, latn, inner, aa = refs
        emb = lat
    else:
        ih, dh, fh, emb, lat, tem, hyp, out, inner, aa = refs
    i = pl.program_id(0)

    @pl.when(i == 0)
    def _prologue():
        hs = (ih, dh, fh)
        for s, (st, sz) in enumerate(_SEGS):
            aa[st:st + sz, :] = jnp.dot(
                emb[st:st + sz, :], hs[s][...], precision=_PREC)
        for s, (st, sz) in enumerate(_SEGS):
            inner[s * _D:(s + 1) * _D, :] = _lk(jax.lax.dot_general(
                aa[st:st + sz, :], lat[st:st + sz, :],
                (((0,), (0,)), ((), ())), precision=_PREC))

    t = jnp.concatenate(
        [jnp.dot(a[...], lat[...], precision=_PREC) for a in adj_ks],
        axis=0)
    t = _lk(t)

    rows = pl.ds(i * _R, _R)
    aa_rows = aa[rows, :]
    b0, b1 = _ISSUE // _R, (_ISSUE + _DEV) // _R
    for s, lo, hi in ((0, 0, b0), (1, b0, b1), (2, b1, _N // _R)):
        @pl.when((i >= lo) & (i < hi))
        def _seg(s=s):
            hyp[...] = _lk(jnp.dot(
                aa_rows, inner[s * _D:(s + 1) * _D, :], precision=_PREC))

    ln = t + hyp[...]
    tem[...] = t
    if first:
        latn[...] = ln
    else:
        out[...] = (emb[rows, :] + lat[rows, :]) + ln


def _adj_specs():
    return [
        pl.BlockSpec((_RS, _N), lambda i, k=k: (_NS * i + k, 0))
        for k in range(_NS)
    ]


_ROW = pl.BlockSpec((_R, _D), lambda i: (i, 0))
_FULL = pl.BlockSpec((_N, _D), lambda i: (0, 0))
_SMALL = pl.BlockSpec((_D, _D), lambda i: (0, 0))


def _scratch():
    return [pltpu.VMEM((3 * _D, _D), jnp.float32),
            pltpu.VMEM((_N, _D), jnp.float32)]


def _layer1(adj, embeds, ih, dh, fh):
    body = lambda *r: _layer_body(True, *r)
    return pl.pallas_call(
        body,
        grid=(_N // _R,),
        in_specs=_adj_specs() + [_SMALL, _SMALL, _SMALL, _FULL],
        out_specs=[_ROW] * 3,
        out_shape=[jax.ShapeDtypeStruct((_N, _D), jnp.float32)] * 3,
        scratch_shapes=_scratch(),
        compiler_params=pltpu.CompilerParams(
            dimension_semantics=("arbitrary",),
        ),
    )(*([adj] * _NS), ih, dh, fh, embeds)


def _layer2(adj, embeds, lat1, ih, dh, fh):
    body = lambda *r: _layer_body(False, *r)
    return pl.pallas_call(
        body,
        grid=(_N // _R,),
        in_specs=_adj_specs() + [_SMALL, _SMALL, _SMALL, _FULL, _FULL],
        out_specs=[_ROW] * 3,
        out_shape=[jax.ShapeDtypeStruct((_N, _D), jnp.float32)] * 3,
        scratch_shapes=_scratch(),
        compiler_params=pltpu.CompilerParams(
            dimension_semantics=("arbitrary",),
        ),
    )(*([adj] * _NS), ih, dh, fh, embeds, lat1)


def kernel(adj, keepRate, iEmbeds, dEmbeds, fEmbeds, iHyper, dHyper, fHyper):
    embeds = jnp.concatenate([iEmbeds, dEmbeds, fEmbeds], axis=0)

    tem1, hyp1, lat1 = _layer1(adj, embeds, iHyper, dHyper, fHyper)
    tem2, hyp2, out = _layer2(adj, embeds, lat1, iHyper, dHyper, fHyper)

    return (out, tem1, tem2, hyp1, hyp2)


# speedup vs baseline: 1.0605x; 1.0605x over previous
import jax
import jax.numpy as jnp
from jax.experimental import pallas as pl
from jax.experimental.pallas import tpu as pltpu

_ISSUE, _DEV, _FILE = 4000, 2000, 4000
_N = _ISSUE + _DEV + _FILE
_D = 128
_LEAKY = 0.1
_NS = 1
_RS = 400
_R = _NS * _RS
_PREC = jax.lax.Precision.DEFAULT
_SEGS = ((0, _ISSUE), (_ISSUE, _DEV), (_ISSUE + _DEV, _FILE))


def _lk(x):
    return jnp.where(x >= 0, x, _LEAKY * x)


def _layer_body(first, *refs):
    adj_ks, refs = refs[:_NS], refs[_NS:]
    if first:
        ie, de, fe, ih, dh, fh, tem, hyp, latn, inner, aa, emb = refs
        lat = emb
    else:
        ie, de, fe, ih, dh, fh, lat, tem, hyp, out, inner, aa, emb = refs
    i = pl.program_id(0)

    @pl.when(i == 0)
    def _prologue():
        es, hs = (ie, de, fe), (ih, dh, fh)
        for s, (st, sz) in enumerate(_SEGS):
            emb[st:st + sz, :] = es[s][...]
            aa[st:st + sz, :] = jnp.dot(
                es[s][...], hs[s][...], precision=_PREC)
        for s, (st, sz) in enumerate(_SEGS):
            inner[s * _D:(s + 1) * _D, :] = _lk(jax.lax.dot_general(
                aa[st:st + sz, :], lat[st:st + sz, :],
                (((0,), (0,)), ((), ())), precision=_PREC))

    t = jnp.concatenate(
        [jnp.dot(a[...], lat[...], precision=_PREC) for a in adj_ks],
        axis=0)
    t = _lk(t)

    rows = pl.ds(i * _R, _R)
    aa_rows = aa[rows, :]
    b0, b1 = _ISSUE // _R, (_ISSUE + _DEV) // _R
    for s, lo, hi in ((0, 0, b0), (1, b0, b1), (2, b1, _N // _R)):
        @pl.when((i >= lo) & (i < hi))
        def _seg(s=s):
            hyp[...] = _lk(jnp.dot(
                aa_rows, inner[s * _D:(s + 1) * _D, :], precision=_PREC))

    ln = t + hyp[...]
    tem[...] = t
    if first:
        latn[...] = ln
    else:
        out[...] = (emb[rows, :] + lat[rows, :]) + ln


def _adj_specs():
    return [
        pl.BlockSpec((_RS, _N), lambda i, k=k: (_NS * i + k, 0))
        for k in range(_NS)
    ]


_ROW = pl.BlockSpec((_R, _D), lambda i: (i, 0))
_FULL = pl.BlockSpec((_N, _D), lambda i: (0, 0))
_SMALL = pl.BlockSpec((_D, _D), lambda i: (0, 0))
_EMB_SPECS = [pl.BlockSpec((sz, _D), lambda i: (0, 0))
              for (_st, sz) in _SEGS]


def _scratch():
    return [pltpu.VMEM((3 * _D, _D), jnp.float32),
            pltpu.VMEM((_N, _D), jnp.float32),
            pltpu.VMEM((_N, _D), jnp.float32)]


def _layer1(adj, ie, de, fe, ih, dh, fh):
    body = lambda *r: _layer_body(True, *r)
    return pl.pallas_call(
        body,
        grid=(_N // _R,),
        in_specs=_adj_specs() + _EMB_SPECS + [_SMALL, _SMALL, _SMALL],
        out_specs=[_ROW] * 3,
        out_shape=[jax.ShapeDtypeStruct((_N, _D), jnp.float32)] * 3,
        scratch_shapes=_scratch(),
        compiler_params=pltpu.CompilerParams(
            dimension_semantics=("arbitrary",),
        ),
    )(*([adj] * _NS), ie, de, fe, ih, dh, fh)


def _layer2(adj, ie, de, fe, lat1, ih, dh, fh):
    body = lambda *r: _layer_body(False, *r)
    return pl.pallas_call(
        body,
        grid=(_N // _R,),
        in_specs=_adj_specs() + _EMB_SPECS + [_SMALL, _SMALL, _SMALL, _FULL],
        out_specs=[_ROW] * 3,
        out_shape=[jax.ShapeDtypeStruct((_N, _D), jnp.float32)] * 3,
        scratch_shapes=_scratch(),
        compiler_params=pltpu.CompilerParams(
            dimension_semantics=("arbitrary",),
        ),
    )(*([adj] * _NS), ie, de, fe, ih, dh, fh, lat1)


def kernel(adj, keepRate, iEmbeds, dEmbeds, fEmbeds, iHyper, dHyper, fHyper):
    tem1, hyp1, lat1 = _layer1(adj, iEmbeds, dEmbeds, fEmbeds,
                               iHyper, dHyper, fHyper)
    tem2, hyp2, out = _layer2(adj, iEmbeds, dEmbeds, fEmbeds, lat1,
                              iHyper, dHyper, fHyper)

    return (out, tem1, tem2, hyp1, hyp2)


# speedup vs baseline: 1.1008x; 1.0380x over previous
import jax
import jax.numpy as jnp
from jax.experimental import pallas as pl
from jax.experimental.pallas import tpu as pltpu

_ISSUE, _DEV, _FILE = 4000, 2000, 4000
_N = _ISSUE + _DEV + _FILE
_D = 128
_LEAKY = 0.1
_R = 400
_NB = _N // _R
_PREC = jax.lax.Precision.DEFAULT
_SEGS = ((0, _ISSUE), (_ISSUE, _DEV), (_ISSUE + _DEV, _FILE))


def _lk(x):
    return jnp.where(x >= 0, x, _LEAKY * x)


def _body(adj, ie, de, fe, ih, dh, fh,
          tem1, hyp1, tem2, hyp2, out, inner, aa, emb, lat1):
    l = pl.program_id(0)
    i = pl.program_id(1)

    @pl.when((l == 0) & (i == 0))
    def _prologue1():
        es, hs = (ie, de, fe), (ih, dh, fh)
        for s, (st, sz) in enumerate(_SEGS):
            emb[st:st + sz, :] = es[s][...]
            aa[st:st + sz, :] = jnp.dot(
                es[s][...], hs[s][...], precision=_PREC)
        for s, (st, sz) in enumerate(_SEGS):
            inner[s * _D:(s + 1) * _D, :] = _lk(jax.lax.dot_general(
                aa[st:st + sz, :], emb[st:st + sz, :],
                (((0,), (0,)), ((), ())), precision=_PREC))

    @pl.when((l == 1) & (i == 0))
    def _prologue2():
        for s, (st, sz) in enumerate(_SEGS):
            inner[s * _D:(s + 1) * _D, :] = _lk(jax.lax.dot_general(
                aa[st:st + sz, :], lat1[st:st + sz, :],
                (((0,), (0,)), ((), ())), precision=_PREC))

    rows = pl.ds(i * _R, _R)
    aa_rows = aa[rows, :]
    b0, b1 = _ISSUE // _R, (_ISSUE + _DEV) // _R
    hyp_scr = jnp.zeros((_R, _D), jnp.float32)
    for s, lo, hi in ((0, 0, b0), (1, b0, b1), (2, b1, _NB)):
        hyp_scr = jnp.where(
            (i >= lo) & (i < hi),
            _lk(jnp.dot(aa_rows, inner[s * _D:(s + 1) * _D, :],
                        precision=_PREC)),
            hyp_scr)

    @pl.when(l == 0)
    def _l1():
        t = _lk(jnp.dot(adj[...], emb[...], precision=_PREC))
        tem1[...] = t
        hyp1[...] = hyp_scr
        lat1[rows, :] = t + hyp_scr

    @pl.when(l == 1)
    def _l2():
        t = _lk(jnp.dot(adj[...], lat1[...], precision=_PREC))
        tem2[...] = t
        hyp2[...] = hyp_scr
        out[...] = (emb[rows, :] + lat1[rows, :]) + (t + hyp_scr)


def kernel(adj, keepRate, iEmbeds, dEmbeds, fEmbeds, iHyper, dHyper, fHyper):
    small = pl.BlockSpec((_D, _D), lambda l, i: (0, 0))
    emb_specs = [pl.BlockSpec((sz, _D), lambda l, i: (0, 0))
                 for (_st, sz) in _SEGS]
    row0 = pl.BlockSpec((_R, _D),
                        lambda l, i: (i * (1 - l) + (_NB - 1) * l, 0))
    row1 = pl.BlockSpec((_R, _D), lambda l, i: (i * l, 0))
    o = jax.ShapeDtypeStruct((_N, _D), jnp.float32)
    tem1, hyp1, tem2, hyp2, out = pl.pallas_call(
        _body,
        grid=(2, _NB),
        in_specs=[pl.BlockSpec((_R, _N), lambda l, i: (i, 0))] + emb_specs
                 + [small] * 3,
        out_specs=[row0, row0, row1, row1, row1],
        out_shape=[o] * 5,
        scratch_shapes=[pltpu.VMEM((3 * _D, _D), jnp.float32),
                        pltpu.VMEM((_N, _D), jnp.float32),
                        pltpu.VMEM((_N, _D), jnp.float32),
                        pltpu.VMEM((_N, _D), jnp.float32)],
        compiler_params=pltpu.CompilerParams(
            dimension_semantics=("arbitrary", "arbitrary"),
        ),
    )(adj, iEmbeds, dEmbeds, fEmbeds, iHyper, dHyper, fHyper)

    return (out, tem1, tem2, hyp1, hyp2)
